# Initial kernel scaffold; baseline (speedup 1.0000x reference)
import jax
import jax.numpy as jnp
from jax.experimental import pallas as pl


def kernel(x, edge_index, W1l, b1l, W1r, W2l, b2l, W2r, Wc, bc):
    raise NotImplementedError("write your pallas kernel here")


# speedup vs baseline: 3.9920x; 3.9920x over previous
import functools

import jax
import jax.numpy as jnp
from jax import lax
from jax.experimental import pallas as pl
from jax.experimental.pallas import tpu as pltpu
from jax.experimental.pallas import tpu_sc as plsc

N_NODES = 10000
N_EDGES = 320000
D = 128

NC = 2
NS = 16

CHUNK = 80
RPT = 624
TAIL = N_NODES - NS * RPT
SROWS = 16
NSTAGE = RPT // SROWS


def _zero_acc(acc, stage, row_lo, is_last):
    zv = jnp.zeros((16,), jnp.float32)
    for i in range(SROWS):
        for j in range(D // 16):
            stage[i, pl.ds(j * 16, 16)] = zv

    @pl.loop(0, NSTAGE)
    def _zero(i):
        pltpu.sync_copy(stage, acc.at[pl.ds(row_lo + i * SROWS, SROWS)])

    @pl.when(is_last)
    def _zero_tail():
        pltpu.sync_copy(stage.at[pl.ds(0, TAIL)],
                        acc.at[pl.ds(NS * RPT, TAIL)])


def _agg1_body(x_hbm, src_hbm, dst_hbm, out_hbm, cnt_hbm,
               acc, ones_v, idx_s, idx_d, rows, stage, gsem):
    c = lax.axis_index("c")
    s = lax.axis_index("s")
    row_lo = s * RPT
    is_last = s == NS - 1

    _zero_acc(acc, stage, row_lo, is_last)
    one = jnp.ones((16,), jnp.float32)
    for i in range(CHUNK):
        for j in range(D // 16):
            ones_v[i, pl.ds(j * 16, 16)] = one
    plsc.subcore_barrier()

    ept = N_EDGES // NS
    ebase = s * ept

    @pl.loop(0, ept // CHUNK)
    def _edge_chunk(j):
        eb = pl.multiple_of(ebase + j * CHUNK, 8)
        pltpu.sync_copy(dst_hbm.at[pl.ds(eb, CHUNK)], idx_d)

        @pl.when(c == 0)
        def _feat():
            pltpu.sync_copy(src_hbm.at[pl.ds(eb, CHUNK)], idx_s)
            pltpu.async_copy(x_hbm.at[idx_s], rows, gsem).wait()
            pltpu.sync_copy(rows, acc.at[idx_d], add=True)

        @pl.when(c == 1)
        def _cnt():
            pltpu.sync_copy(ones_v, acc.at[idx_d], add=True)

    plsc.subcore_barrier()

    @pl.loop(0, NSTAGE)
    def _copy_out(i):
        lo = row_lo + i * SROWS
        pltpu.sync_copy(acc.at[pl.ds(lo, SROWS)], stage)

        @pl.when(c == 0)
        def _o0():
            pltpu.sync_copy(stage, out_hbm.at[pl.ds(lo, SROWS)])

        @pl.when(c == 1)
        def _o1():
            pltpu.sync_copy(stage, cnt_hbm.at[pl.ds(lo, SROWS)])

    @pl.when(is_last)
    def _copy_tail():
        pltpu.sync_copy(acc.at[pl.ds(NS * RPT, TAIL)], stage.at[pl.ds(0, TAIL)])

        @pl.when(c == 0)
        def _t0():
            pltpu.sync_copy(stage.at[pl.ds(0, TAIL)],
                            out_hbm.at[pl.ds(NS * RPT, TAIL)])

        @pl.when(c == 1)
        def _t1():
            pltpu.sync_copy(stage.at[pl.ds(0, TAIL)],
                            cnt_hbm.at[pl.ds(NS * RPT, TAIL)])


def _agg2_body(x_hbm, src_hbm, dst_hbm, out_hbm,
               acc, idx_s, idx_d, rows, stage, gsem):
    c = lax.axis_index("c")
    s = lax.axis_index("s")
    row_lo = s * RPT
    is_last = s == NS - 1

    _zero_acc(acc, stage, row_lo, is_last)
    plsc.subcore_barrier()

    epc = N_EDGES // NC
    ept = N_EDGES // (NC * NS)
    ebase = c * epc + s * ept

    @pl.loop(0, ept // CHUNK)
    def _edge_chunk(j):
        eb = pl.multiple_of(ebase + j * CHUNK, 8)
        pltpu.sync_copy(src_hbm.at[pl.ds(eb, CHUNK)], idx_s)
        pltpu.sync_copy(dst_hbm.at[pl.ds(eb, CHUNK)], idx_d)
        pltpu.async_copy(x_hbm.at[idx_s], rows, gsem).wait()
        pltpu.sync_copy(rows, acc.at[idx_d], add=True)

    plsc.subcore_barrier()

    @pl.loop(0, NSTAGE)
    def _copy_out(i):
        lo = row_lo + i * SROWS
        pltpu.sync_copy(acc.at[pl.ds(lo, SROWS)], stage)
        pltpu.sync_copy(stage, out_hbm.at[c, pl.ds(lo, SROWS)])

    @pl.when(is_last)
    def _copy_tail():
        pltpu.sync_copy(acc.at[pl.ds(NS * RPT, TAIL)], stage.at[pl.ds(0, TAIL)])
        pltpu.sync_copy(stage.at[pl.ds(0, TAIL)],
                        out_hbm.at[c, pl.ds(NS * RPT, TAIL)])


_MESH = plsc.VectorSubcoreMesh(core_axis_name="c", subcore_axis_name="s")

_agg1 = pl.kernel(
    _agg1_body,
    out_type=(jax.ShapeDtypeStruct((N_NODES, D), jnp.float32),
              jax.ShapeDtypeStruct((N_NODES, D), jnp.float32)),
    mesh=_MESH,
    scratch_types=[
        pltpu.VMEM_SHARED((N_NODES, D), jnp.float32),
        pltpu.VMEM((CHUNK, D), jnp.float32),
        pltpu.VMEM((CHUNK,), jnp.int32),
        pltpu.VMEM((CHUNK,), jnp.int32),
        pltpu.VMEM((CHUNK, D), jnp.float32),
        pltpu.VMEM((SROWS, D), jnp.float32),
        pltpu.SemaphoreType.DMA,
    ],
)

_agg2 = pl.kernel(
    _agg2_body,
    out_type=jax.ShapeDtypeStruct((NC, N_NODES, D), jnp.float32),
    mesh=_MESH,
    scratch_types=[
        pltpu.VMEM_SHARED((N_NODES, D), jnp.float32),
        pltpu.VMEM((CHUNK,), jnp.int32),
        pltpu.VMEM((CHUNK,), jnp.int32),
        pltpu.VMEM((CHUNK, D), jnp.float32),
        pltpu.VMEM((SROWS, D), jnp.float32),
        pltpu.SemaphoreType.DMA,
    ],
)


def _tc1_body(agg_ref, cnt_ref, x_ref, wl_ref, bl_ref, wr_ref, o_ref):
    recip = 1.0 / jnp.maximum(cnt_ref[:, :1], 1.0)
    mean = agg_ref[...] * recip
    h = (jnp.dot(mean, wl_ref[...], preferred_element_type=jnp.float32)
         + bl_ref[...]
         + jnp.dot(x_ref[...], wr_ref[...], preferred_element_type=jnp.float32))
    o_ref[...] = jnp.maximum(h, 0.0)


def _tc2_body(p_ref, cnt_ref, h_ref, wl_ref, bl_ref, wr_ref, wc_ref, bc_ref,
              o_ref):
    agg = p_ref[0] + p_ref[1]
    recip = 1.0 / jnp.maximum(cnt_ref[:, :1], 1.0)
    mean = agg * recip
    h = (jnp.dot(mean, wl_ref[...], preferred_element_type=jnp.float32)
         + bl_ref[...]
         + jnp.dot(h_ref[...], wr_ref[...], preferred_element_type=jnp.float32))
    h = jnp.maximum(h, 0.0)
    o = jnp.dot(h, wc_ref[...], preferred_element_type=jnp.float32) + bc_ref[...]
    m = jnp.max(o, axis=1, keepdims=True)
    lse = m + jnp.log(jnp.sum(jnp.exp(o - m), axis=1, keepdims=True))
    o_ref[...] = o - lse


_RB = 2000


def _full(shape):
    return pl.BlockSpec(shape, lambda i: tuple(0 for _ in shape))


def _tc1(agg, cnt, x, wl, bl, wr):
    return pl.pallas_call(
        _tc1_body,
        grid=(N_NODES // _RB,),
        in_specs=[
            pl.BlockSpec((_RB, D), lambda i: (i, 0)),
            pl.BlockSpec((_RB, D), lambda i: (i, 0)),
            pl.BlockSpec((_RB, D), lambda i: (i, 0)),
            _full((D, D)), _full((1, D)), _full((D, D)),
        ],
        out_specs=pl.BlockSpec((_RB, D), lambda i: (i, 0)),
        out_shape=jax.ShapeDtypeStruct((N_NODES, D), jnp.float32),
    )(agg, cnt, x, wl, bl, wr)


def _tc2(p, cnt, h, wl, bl, wr, wc, bc):
    return pl.pallas_call(
        _tc2_body,
        grid=(N_NODES // _RB,),
        in_specs=[
            pl.BlockSpec((NC, _RB, D), lambda i: (0, i, 0)),
            pl.BlockSpec((_RB, D), lambda i: (i, 0)),
            pl.BlockSpec((_RB, D), lambda i: (i, 0)),
            _full((D, D)), _full((1, D)), _full((D, D)),
            _full((D, D)), _full((1, D)),
        ],
        out_specs=pl.BlockSpec((_RB, D), lambda i: (i, 0)),
        out_shape=jax.ShapeDtypeStruct((N_NODES, D), jnp.float32),
    )(p, cnt, h, wl, bl, wr, wc, bc)


def kernel(x, edge_index, W1l, b1l, W1r, W2l, b2l, W2r, Wc, bc):
    src = edge_index[0].astype(jnp.int32)
    dst = edge_index[1].astype(jnp.int32)

    agg1, cnt = _agg1(x, src, dst)
    h1 = _tc1(agg1, cnt, x, W1l.T, b1l[None, :], W1r.T)
    p2 = _agg2(h1, src, dst)
    out = _tc2(p2, cnt, h1, W2l.T, b2l[None, :], W2r.T, Wc.T, bc[None, :])
    return out


# speedup vs baseline: 6.5663x; 1.6449x over previous
import functools

import jax
import jax.numpy as jnp
from jax import lax
from jax.experimental import pallas as pl
from jax.experimental.pallas import tpu as pltpu
from jax.experimental.pallas import tpu_sc as plsc

N_NODES = 10000
N_EDGES = 320000
D = 128

NC = 2
NS = 16

CHUNK = 80
RPT = 624
TAIL = N_NODES - NS * RPT
SROWS = 16
NSTAGE = RPT // SROWS


def _zero_acc(acc, stage, row_lo, is_last):
    zv = jnp.zeros((16,), jnp.float32)
    for i in range(SROWS):
        for j in range(D // 16):
            stage[i, pl.ds(j * 16, 16)] = zv

    @pl.loop(0, NSTAGE)
    def _zero(i):
        pltpu.sync_copy(stage, acc.at[pl.ds(row_lo + i * SROWS, SROWS)])

    @pl.when(is_last)
    def _zero_tail():
        pltpu.sync_copy(stage.at[pl.ds(0, TAIL)],
                        acc.at[pl.ds(NS * RPT, TAIL)])


def _pipelined_edges(nchunk, start, finish):
    start(0, 0)
    npairs = nchunk // 2

    @pl.loop(0, npairs)
    def _pair(t):
        c0 = 2 * t
        start(c0 + 1, 1)
        finish(0)

        @pl.when(c0 + 2 < nchunk)
        def _nxt():
            start(c0 + 2, 0)

        finish(1)

    if nchunk % 2:
        finish(0)


def _agg1_body(x_hbm, src_hbm, dst_hbm, out_hbm, cnt_hbm,
               acc, ones_v, idx_s0, idx_s1, idx_d0, idx_d1,
               rows0, rows1, stage, gsem0, gsem1):
    c = lax.axis_index("c")
    s = lax.axis_index("s")
    row_lo = s * RPT
    is_last = s == NS - 1
    idx_s = (idx_s0, idx_s1)
    idx_d = (idx_d0, idx_d1)
    rows = (rows0, rows1)
    gsem = (gsem0, gsem1)

    _zero_acc(acc, stage, row_lo, is_last)
    one = jnp.ones((16,), jnp.float32)
    for i in range(CHUNK):
        for j in range(D // 16):
            ones_v[i, pl.ds(j * 16, 16)] = one
    plsc.subcore_barrier()

    ept = N_EDGES // NS
    ebase = s * ept

    def start(j, b):
        eb = pl.multiple_of(ebase + j * CHUNK, 8)
        pltpu.sync_copy(dst_hbm.at[pl.ds(eb, CHUNK)], idx_d[b])

        @pl.when(c == 0)
        def _():
            pltpu.sync_copy(src_hbm.at[pl.ds(eb, CHUNK)], idx_s[b])
            pltpu.make_async_copy(x_hbm.at[idx_s[b]], rows[b],
                                  gsem[b]).start()

    def finish(b):
        @pl.when(c == 0)
        def _():
            pltpu.make_async_copy(x_hbm.at[idx_s[b]], rows[b],
                                  gsem[b]).wait()
            pltpu.sync_copy(rows[b], acc.at[idx_d[b]], add=True)

        @pl.when(c == 1)
        def _():
            pltpu.sync_copy(ones_v, acc.at[idx_d[b]], add=True)

    _pipelined_edges(ept // CHUNK, start, finish)
    plsc.subcore_barrier()

    @pl.loop(0, NSTAGE)
    def _copy_out(i):
        lo = row_lo + i * SROWS
        pltpu.sync_copy(acc.at[pl.ds(lo, SROWS)], stage)

        @pl.when(c == 0)
        def _o0():
            pltpu.sync_copy(stage, out_hbm.at[pl.ds(lo, SROWS)])

        @pl.when(c == 1)
        def _o1():
            pltpu.sync_copy(stage, cnt_hbm.at[pl.ds(lo, SROWS)])

    @pl.when(is_last)
    def _copy_tail():
        pltpu.sync_copy(acc.at[pl.ds(NS * RPT, TAIL)], stage.at[pl.ds(0, TAIL)])

        @pl.when(c == 0)
        def _t0():
            pltpu.sync_copy(stage.at[pl.ds(0, TAIL)],
                            out_hbm.at[pl.ds(NS * RPT, TAIL)])

        @pl.when(c == 1)
        def _t1():
            pltpu.sync_copy(stage.at[pl.ds(0, TAIL)],
                            cnt_hbm.at[pl.ds(NS * RPT, TAIL)])


def _agg2_body(x_hbm, src_hbm, dst_hbm, out_hbm,
               acc, idx_s0, idx_s1, idx_d0, idx_d1,
               rows0, rows1, stage, gsem0, gsem1):
    c = lax.axis_index("c")
    s = lax.axis_index("s")
    row_lo = s * RPT
    is_last = s == NS - 1
    idx_s = (idx_s0, idx_s1)
    idx_d = (idx_d0, idx_d1)
    rows = (rows0, rows1)
    gsem = (gsem0, gsem1)

    _zero_acc(acc, stage, row_lo, is_last)
    plsc.subcore_barrier()

    epc = N_EDGES // NC
    ept = N_EDGES // (NC * NS)
    ebase = c * epc + s * ept

    def start(j, b):
        eb = pl.multiple_of(ebase + j * CHUNK, 8)
        pltpu.sync_copy(src_hbm.at[pl.ds(eb, CHUNK)], idx_s[b])
        pltpu.sync_copy(dst_hbm.at[pl.ds(eb, CHUNK)], idx_d[b])
        pltpu.make_async_copy(x_hbm.at[idx_s[b]], rows[b], gsem[b]).start()

    def finish(b):
        pltpu.make_async_copy(x_hbm.at[idx_s[b]], rows[b], gsem[b]).wait()
        pltpu.sync_copy(rows[b], acc.at[idx_d[b]], add=True)

    _pipelined_edges(ept // CHUNK, start, finish)
    plsc.subcore_barrier()

    @pl.loop(0, NSTAGE)
    def _copy_out(i):
        lo = row_lo + i * SROWS
        pltpu.sync_copy(acc.at[pl.ds(lo, SROWS)], stage)
        pltpu.sync_copy(stage, out_hbm.at[c, pl.ds(lo, SROWS)])

    @pl.when(is_last)
    def _copy_tail():
        pltpu.sync_copy(acc.at[pl.ds(NS * RPT, TAIL)], stage.at[pl.ds(0, TAIL)])
        pltpu.sync_copy(stage.at[pl.ds(0, TAIL)],
                        out_hbm.at[c, pl.ds(NS * RPT, TAIL)])


_MESH = plsc.VectorSubcoreMesh(core_axis_name="c", subcore_axis_name="s")

_agg1 = pl.kernel(
    _agg1_body,
    out_type=(jax.ShapeDtypeStruct((N_NODES, D), jnp.float32),
              jax.ShapeDtypeStruct((N_NODES, D), jnp.float32)),
    mesh=_MESH,
    scratch_types=[
        pltpu.VMEM_SHARED((N_NODES, D), jnp.float32),
        pltpu.VMEM((CHUNK, D), jnp.float32),
        pltpu.VMEM((CHUNK,), jnp.int32),
        pltpu.VMEM((CHUNK,), jnp.int32),
        pltpu.VMEM((CHUNK,), jnp.int32),
        pltpu.VMEM((CHUNK,), jnp.int32),
        pltpu.VMEM((CHUNK, D), jnp.float32),
        pltpu.VMEM((CHUNK, D), jnp.float32),
        pltpu.VMEM((SROWS, D), jnp.float32),
        pltpu.SemaphoreType.DMA,
        pltpu.SemaphoreType.DMA,
    ],
)

_agg2 = pl.kernel(
    _agg2_body,
    out_type=jax.ShapeDtypeStruct((NC, N_NODES, D), jnp.float32),
    mesh=_MESH,
    scratch_types=[
        pltpu.VMEM_SHARED((N_NODES, D), jnp.float32),
        pltpu.VMEM((CHUNK,), jnp.int32),
        pltpu.VMEM((CHUNK,), jnp.int32),
        pltpu.VMEM((CHUNK,), jnp.int32),
        pltpu.VMEM((CHUNK,), jnp.int32),
        pltpu.VMEM((CHUNK, D), jnp.float32),
        pltpu.VMEM((CHUNK, D), jnp.float32),
        pltpu.VMEM((SROWS, D), jnp.float32),
        pltpu.SemaphoreType.DMA,
        pltpu.SemaphoreType.DMA,
    ],
)


def _tc1_body(agg_ref, cnt_ref, x_ref, wl_ref, bl_ref, wr_ref, o_ref):
    recip = 1.0 / jnp.maximum(cnt_ref[:, :1], 1.0)
    mean = agg_ref[...] * recip
    h = (jnp.dot(mean, wl_ref[...], preferred_element_type=jnp.float32)
         + bl_ref[...]
         + jnp.dot(x_ref[...], wr_ref[...], preferred_element_type=jnp.float32))
    o_ref[...] = jnp.maximum(h, 0.0)


def _tc2_body(p_ref, cnt_ref, h_ref, wl_ref, bl_ref, wr_ref, wc_ref, bc_ref,
              o_ref):
    agg = p_ref[0] + p_ref[1]
    recip = 1.0 / jnp.maximum(cnt_ref[:, :1], 1.0)
    mean = agg * recip
    h = (jnp.dot(mean, wl_ref[...], preferred_element_type=jnp.float32)
         + bl_ref[...]
         + jnp.dot(h_ref[...], wr_ref[...], preferred_element_type=jnp.float32))
    h = jnp.maximum(h, 0.0)
    o = jnp.dot(h, wc_ref[...], preferred_element_type=jnp.float32) + bc_ref[...]
    m = jnp.max(o, axis=1, keepdims=True)
    lse = m + jnp.log(jnp.sum(jnp.exp(o - m), axis=1, keepdims=True))
    o_ref[...] = o - lse


_RB = 2000


def _full(shape):
    return pl.BlockSpec(shape, lambda i: tuple(0 for _ in shape))


def _tc1(agg, cnt, x, wl, bl, wr):
    return pl.pallas_call(
        _tc1_body,
        grid=(N_NODES // _RB,),
        in_specs=[
            pl.BlockSpec((_RB, D), lambda i: (i, 0)),
            pl.BlockSpec((_RB, D), lambda i: (i, 0)),
            pl.BlockSpec((_RB, D), lambda i: (i, 0)),
            _full((D, D)), _full((1, D)), _full((D, D)),
        ],
        out_specs=pl.BlockSpec((_RB, D), lambda i: (i, 0)),
        out_shape=jax.ShapeDtypeStruct((N_NODES, D), jnp.float32),
    )(agg, cnt, x, wl, bl, wr)


def _tc2(p, cnt, h, wl, bl, wr, wc, bc):
    return pl.pallas_call(
        _tc2_body,
        grid=(N_NODES // _RB,),
        in_specs=[
            pl.BlockSpec((NC, _RB, D), lambda i: (0, i, 0)),
            pl.BlockSpec((_RB, D), lambda i: (i, 0)),
            pl.BlockSpec((_RB, D), lambda i: (i, 0)),
            _full((D, D)), _full((1, D)), _full((D, D)),
            _full((D, D)), _full((1, D)),
        ],
        out_specs=pl.BlockSpec((_RB, D), lambda i: (i, 0)),
        out_shape=jax.ShapeDtypeStruct((N_NODES, D), jnp.float32),
    )(p, cnt, h, wl, bl, wr, wc, bc)


def kernel(x, edge_index, W1l, b1l, W1r, W2l, b2l, W2r, Wc, bc):
    src = edge_index[0].astype(jnp.int32)
    dst = edge_index[1].astype(jnp.int32)

    agg1, cnt = _agg1(x, src, dst)
    h1 = _tc1(agg1, cnt, x, W1l.T, b1l[None, :], W1r.T)
    p2 = _agg2(h1, src, dst)
    out = _tc2(p2, cnt, h1, W2l.T, b2l[None, :], W2r.T, Wc.T, bc[None, :])
    return out
